# baseline (speedup 1.0000x reference)
import jax
import jax.numpy as jnp
from jax.experimental import pallas as pl
from jax.experimental.pallas import tpu as pltpu

_NPIECE = 4


def _body(x_hbm, cb_ref, out_hbm, x_v, out_v, sem_x, sem_o):
    M, D = x_v.shape
    K = cb_ref.shape[0]
    H = M // _NPIECE
    cpx = []
    for h in range(_NPIECE):
        rows = pl.ds(h * H, H)
        cp = pltpu.make_async_copy(x_hbm.at[rows], x_v.at[rows], sem_x.at[h])
        cp.start()
        cpx.append(cp)
    cb = cb_ref[...]
    c2 = jnp.sum(cb * cb, axis=1)[None, :]
    cpo = []
    for h in range(_NPIECE):
        rows = pl.ds(h * H, H)
        cpx[h].wait()
        xv = x_v[rows, :]
        x2 = jnp.sum(xv * xv, axis=1, keepdims=True)
        xc = jax.lax.dot_general(xv, cb, (((1,), (1,)), ((), ())),
                                 preferred_element_type=jnp.float32)
        dist = x2 + c2 - 2.0 * xc
        mins = jnp.min(dist, axis=1, keepdims=True)
        kio = (jax.lax.broadcasted_iota(jnp.int32, (H, K), 1)
               .astype(jnp.float32))
        idx = jnp.min(jnp.where(dist == mins, kio, jnp.float32(K)),
                      axis=1, keepdims=True)
        onehot = jnp.where(kio == idx, jnp.float32(1), jnp.float32(0))
        out_v[rows, :] = jax.lax.dot_general(
            onehot, cb, (((1,), (0,)), ((), ())),
            preferred_element_type=jnp.float32)
        cp = pltpu.make_async_copy(out_v.at[rows], out_hbm.at[rows],
                                   sem_o.at[h])
        cp.start()
        cpo.append(cp)
    for cp in cpo:
        cp.wait()


def kernel(x, codebook):
    N, C, H, W = x.shape
    K = codebook.shape[0]
    D = H * W
    M = N * C
    x_flat = x.reshape(M, D)
    cb_flat = codebook.reshape(K, D)
    out = pl.pallas_call(
        _body,
        in_specs=[pl.BlockSpec(memory_space=pl.ANY),
                  pl.BlockSpec((K, D), lambda: (0, 0))],
        out_specs=pl.BlockSpec(memory_space=pl.ANY),
        out_shape=jax.ShapeDtypeStruct((M, D), jnp.float32),
        scratch_shapes=[
            pltpu.VMEM((M, D), jnp.float32),
            pltpu.VMEM((M, D), jnp.float32),
            pltpu.SemaphoreType.DMA((_NPIECE,)),
            pltpu.SemaphoreType.DMA((_NPIECE,)),
        ],
    )(x_flat, cb_flat)
    return out.reshape(N, C, H, W)


# speedup vs baseline: 1.1935x; 1.1935x over previous
import jax
import jax.numpy as jnp
from jax.experimental import pallas as pl
from jax.experimental.pallas import tpu as pltpu


def _body(x_ref, cb_ref, out_hbm, out_v, sem_o):
    M, D = x_ref.shape
    K = cb_ref.shape[0]
    H = M // 8
    xv = x_ref[...]
    cb = cb_ref[...]
    x2 = jnp.sum(xv * xv, axis=1, keepdims=True)
    c2 = jnp.sum(cb * cb, axis=1)[None, :]
    xc = jax.lax.dot_general(xv, cb, (((1,), (1,)), ((), ())),
                             preferred_element_type=jnp.float32)
    dist = x2 + c2 - 2.0 * xc
    mins = jnp.min(dist, axis=1, keepdims=True)
    kio = jax.lax.broadcasted_iota(jnp.int32, (M, K), 1).astype(jnp.float32)
    idx = jnp.min(jnp.where(dist == mins, kio, jnp.float32(K)),
                  axis=1, keepdims=True)
    onehot = jnp.where(kio == idx, jnp.float32(1), jnp.float32(0))
    cps = []
    for h in range(8):
        rows = pl.ds(h * H, H)
        out_v[rows, :] = jax.lax.dot_general(
            onehot[h * H:(h + 1) * H, :], cb, (((1,), (0,)), ((), ())),
            preferred_element_type=jnp.float32)
        cp = pltpu.make_async_copy(out_v.at[rows], out_hbm.at[rows],
                                   sem_o.at[h])
        cp.start()
        cps.append(cp)
    for cp in cps:
        cp.wait()


def kernel(x, codebook):
    N, C, H, W = x.shape
    K = codebook.shape[0]
    D = H * W
    M = N * C
    x_flat = x.reshape(M, D)
    cb_flat = codebook.reshape(K, D)
    out = pl.pallas_call(
        _body,
        out_specs=pl.BlockSpec(memory_space=pl.ANY),
        out_shape=jax.ShapeDtypeStruct((M, D), jnp.float32),
        scratch_shapes=[
            pltpu.VMEM((M, D), jnp.float32),
            pltpu.SemaphoreType.DMA((8,)),
        ],
    )(x_flat, cb_flat)
    return out.reshape(N, C, H, W)


# speedup vs baseline: 1.1973x; 1.0032x over previous
import jax
import jax.numpy as jnp
from jax.experimental import pallas as pl
from jax.experimental.pallas import tpu as pltpu


def _body(x_ref, cb_ref, out_hbm, out_v, sem_o):
    M, D = x_ref.shape
    K = cb_ref.shape[0]
    H = M // 4
    xv = x_ref[...]
    cb = cb_ref[...]
    x2 = jnp.sum(xv * xv, axis=1, keepdims=True)
    c2 = jnp.sum(cb * cb, axis=1)[None, :]
    xc = jax.lax.dot_general(xv, cb, (((1,), (1,)), ((), ())),
                             preferred_element_type=jnp.float32)
    dist = x2 + c2 - 2.0 * xc
    mins = jnp.min(dist, axis=1, keepdims=True)
    kio = jax.lax.broadcasted_iota(jnp.int32, (M, K), 1).astype(jnp.float32)
    idx = jnp.min(jnp.where(dist == mins, kio, jnp.float32(K)),
                  axis=1, keepdims=True)
    onehot = jnp.where(kio == idx, jnp.float32(1), jnp.float32(0))
    cps = []
    for h in range(4):
        rows = pl.ds(h * H, H)
        out_v[rows, :] = jax.lax.dot_general(
            onehot[h * H:(h + 1) * H, :], cb, (((1,), (0,)), ((), ())),
            preferred_element_type=jnp.float32)
        cp = pltpu.make_async_copy(out_v.at[rows], out_hbm.at[rows],
                                   sem_o.at[h])
        cp.start()
        cps.append(cp)
    for cp in cps:
        cp.wait()


def kernel(x, codebook):
    N, C, H, W = x.shape
    K = codebook.shape[0]
    D = H * W
    M = N * C
    x_flat = x.reshape(M, D)
    cb_flat = codebook.reshape(K, D)
    out = pl.pallas_call(
        _body,
        out_specs=pl.BlockSpec(memory_space=pl.ANY),
        out_shape=jax.ShapeDtypeStruct((M, D), jnp.float32),
        scratch_shapes=[
            pltpu.VMEM((M, D), jnp.float32),
            pltpu.SemaphoreType.DMA((4,)),
        ],
    )(x_flat, cb_flat)
    return out.reshape(N, C, H, W)
